# Initial kernel scaffold; baseline (speedup 1.0000x reference)
import jax
import jax.numpy as jnp
from jax.experimental import pallas as pl


def kernel(x, edge_index, batch, target, gat1_W, gat1_as, gat1_ad, gat1_b, gat2_W, gat2_as, gat2_ad, gat2_b, fcg_W, fcg_b, emb, c1_W, c1_b, c2_W, c2_b, c3_W, c3_b, fcp_W, fcp_b, fc1_W, fc1_b, fc2_W, fc2_b, fc3_W, fc3_b, outW, outb):
    raise NotImplementedError("write your pallas kernel here")


# speedup vs baseline: 1.0381x; 1.0381x over previous
import jax
import jax.numpy as jnp
import numpy as np
from jax.experimental import pallas as pl
from jax.experimental.pallas import tpu as pltpu

N = 10000
E = 160000
F_IN = 78
H1 = 10
C1 = 78
OUT_DIM = 128
B = 128
L = 1000
VOCAB = 27
EMB = 128


def _head_body(xc_ref, w1, b1, w2, b2, w3, b3, w4, b4, o_ref):
    h = jnp.dot(xc_ref[...], w1[...], preferred_element_type=jnp.float32)
    h = jnp.maximum(h + b1[...], 0.0)
    h = jnp.dot(h, w2[...], preferred_element_type=jnp.float32)
    h = jnp.maximum(h + b2[...], 0.0)
    h = jnp.dot(h, w3[...], preferred_element_type=jnp.float32)
    h = jnp.maximum(h + b3[...], 0.0)
    o_ref[...] = jnp.dot(h, w4[...], preferred_element_type=jnp.float32) + b4[...]


def _mlp_head(xc, fc1_W, fc1_b, fc2_W, fc2_b, fc3_W, fc3_b, outW, outb):
    return pl.pallas_call(
        _head_body,
        out_shape=jax.ShapeDtypeStruct((B, 1), jnp.float32),
    )(xc, fc1_W, fc1_b.reshape(1, -1), fc2_W, fc2_b.reshape(1, -1),
      fc3_W, fc3_b.reshape(1, -1), outW, outb.reshape(1, -1))


def _gat(x, ei, W, a_s, a_d, b, H, C):
    n = x.shape[0]
    xp = (x @ W).reshape(n, H, C)
    src = ei[0]
    dst = ei[1]
    a_src = (xp * a_s[None]).sum(-1)
    a_dst = (xp * a_d[None]).sum(-1)
    e = jax.nn.leaky_relu(a_src[src] + a_dst[dst], 0.2)
    m = jax.ops.segment_max(e, dst, num_segments=n)
    m = jnp.where(jnp.isfinite(m), m, 0.0)
    ex = jnp.exp(e - m[dst])
    s = jax.ops.segment_sum(ex, dst, num_segments=n)
    alpha = ex / (s[dst] + 1e-16)
    out = jax.ops.segment_sum(alpha[:, :, None] * xp[src], dst, num_segments=n)
    return out.reshape(n, H * C) + b


def _conv1d(x, W, b):
    y = jax.lax.conv_general_dilated(x, W, (1,), 'VALID',
                                     dimension_numbers=('NCH', 'OIH', 'NCH'))
    return y + b[None, :, None]


def kernel(x, edge_index, batch, target, gat1_W, gat1_as, gat1_ad, gat1_b,
           gat2_W, gat2_as, gat2_ad, gat2_b, fcg_W, fcg_b, emb, c1_W, c1_b,
           c2_W, c2_b, c3_W, c3_b, fcp_W, fcp_b, fc1_W, fc1_b, fc2_W, fc2_b,
           fc3_W, fc3_b, outW, outb):
    n = x.shape[0]
    loops = jnp.arange(n, dtype=edge_index.dtype)
    ei = jnp.concatenate([edge_index, jnp.stack([loops, loops])], axis=1)
    h = jax.nn.elu(_gat(x, ei, gat1_W, gat1_as, gat1_ad, gat1_b, H1, C1))
    h = jax.nn.relu(_gat(h, ei, gat2_W, gat2_as, gat2_ad, gat2_b, 1, OUT_DIM))
    g = jax.ops.segment_max(h, batch, num_segments=B)
    g = jnp.where(jnp.isfinite(g), g, 0.0)
    g = jax.nn.relu(g @ fcg_W + fcg_b)
    t = emb[target]
    t = jnp.transpose(t, (0, 2, 1))
    t = jax.nn.relu(_conv1d(t, c1_W, c1_b))
    t = jax.nn.relu(_conv1d(t, c2_W, c2_b))
    t = jax.nn.relu(_conv1d(t, c3_W, c3_b))
    t = jnp.max(t, axis=2)
    t = jax.nn.relu(t @ fcp_W + fcp_b)
    xc = jnp.concatenate([g, t], axis=1)
    return _mlp_head(xc, fc1_W, fc1_b, fc2_W, fc2_b, fc3_W, fc3_b, outW, outb)
